# baseline (speedup 1.0000x reference)
import functools

import jax
import jax.numpy as jnp
from jax import lax
from jax.experimental import pallas as pl
from jax.experimental.pallas import tpu as pltpu
from jax.experimental.pallas import tpu_sc as plsc

_ALPHA = 0.6
_BETA = 0.25
_GAMMA = 0.15
_TOPK = 10

_ROWS = 4096
_COLS = 2048
_LANES = 16
_NC = 2
_NS = 16
_NW = _NC * _NS
_ROWS_PER_W = _ROWS // _NW
_ROWS_PER_BLK = 16
_NBLK = _ROWS_PER_W // _ROWS_PER_BLK
_CHUNKS = _COLS // _LANES
_RI = 8
_OCOLS = _ROWS_PER_BLK * _LANES

_BLOCK_ROWS = 256
_GRID = _ROWS // _BLOCK_ROWS
_SIG_CHUNKS = 32
_CBLK = 512
_CGRID = _COLS // _CBLK


def _sigmoid16(x):
    return 1.0 / (1.0 + jnp.exp(-x))


def _sc_body(cs_hbm, tk_out, sig_out, buf, obuf, sem0, sem1):
    wid = lax.axis_index("s") * _NC + lax.axis_index("c")
    row0 = wid * _ROWS_PER_W

    sems = (sem0, sem1)
    neg_inf = jnp.full((_LANES,), -jnp.inf, dtype=jnp.float32)
    sig_acc = jnp.zeros((_LANES,), dtype=jnp.float32)

    copies = [None, None]
    copies[0] = pltpu.async_copy(
        cs_hbm.at[pl.ds(row0, _ROWS_PER_BLK)], buf.at[0], sems[0])
    for blk in range(_NBLK):
        cur = blk % 2
        copies[cur].wait()
        if blk + 1 < _NBLK:
            copies[1 - cur] = pltpu.async_copy(
                cs_hbm.at[pl.ds(row0 + (blk + 1) * _ROWS_PER_BLK,
                                _ROWS_PER_BLK)],
                buf.at[1 - cur], sems[1 - cur])
        for rg in range(_ROWS_PER_BLK // _RI):
            def chunk_step_sig(c, carry):
                ts = carry[:_RI]
                sacc = carry[_RI]
                new_ts = []
                for r in range(_RI):
                    v = buf[cur, rg * _RI + r, pl.ds(c * _LANES, _LANES)]
                    sacc = sacc + _sigmoid16(v)
                    vd, _ = plsc.sort_key_val(v, v, descending=True)
                    tb = jnp.maximum(ts[r], vd)
                    ta, _ = plsc.sort_key_val(tb, tb, descending=False)
                    new_ts.append(ta)
                return tuple(new_ts) + (sacc,)

            def chunk_step(c, ts):
                new_ts = []
                for r in range(_RI):
                    v = buf[cur, rg * _RI + r, pl.ds(c * _LANES, _LANES)]
                    vd, _ = plsc.sort_key_val(v, v, descending=True)
                    tb = jnp.maximum(ts[r], vd)
                    ta, _ = plsc.sort_key_val(tb, tb, descending=False)
                    new_ts.append(ta)
                return tuple(new_ts)

            mid = lax.fori_loop(
                0, _SIG_CHUNKS, chunk_step_sig,
                tuple(neg_inf for _ in range(_RI)) + (sig_acc,))
            sig_acc = mid[_RI]
            out = lax.fori_loop(_SIG_CHUNKS, _CHUNKS, chunk_step, mid[:_RI])
            for r in range(_RI):
                row_local = blk * _ROWS_PER_BLK + rg * _RI + r
                obuf[row_local // _ROWS_PER_BLK,
                     pl.ds((row_local % _ROWS_PER_BLK) * _LANES,
                           _LANES)] = out[r]

    pltpu.sync_copy(obuf, tk_out.at[pl.ds(wid * _NBLK, _NBLK)])
    buf[0, 0, pl.ds(0, _LANES)] = sig_acc
    pltpu.sync_copy(buf.at[0, 0, pl.ds(0, _LANES)], sig_out.at[wid])


@functools.cache
def _sc_topk():
    return pl.kernel(
        _sc_body,
        out_type=[
            jax.ShapeDtypeStruct((_ROWS // _ROWS_PER_BLK, _OCOLS),
                                 jnp.float32),
            jax.ShapeDtypeStruct((_NW, _LANES), jnp.float32),
        ],
        mesh=plsc.VectorSubcoreMesh(
            core_axis_name="c", subcore_axis_name="s",
            num_cores=_NC, num_subcores=_NS),
        scratch_types=[
            pltpu.VMEM((2, _ROWS_PER_BLK, _COLS), jnp.float32),
            pltpu.VMEM((_NBLK, _OCOLS), jnp.float32),
            pltpu.SemaphoreType.DMA,
            pltpu.SemaphoreType.DMA,
        ],
        compiler_params=pltpu.CompilerParams(
            needs_layout_passes=False, use_tc_tiling_on_sc=True),
    )


def _tc_dense_body(dl_ref, lb_ref, cs_ref, bce_ref, sig_ref):
    i = pl.program_id(0)
    j = pl.program_id(1)

    @pl.when((i == 0) & (j == 0))
    def _init():
        bce_ref[0, 0] = 0.0
        sig_ref[0, 0] = 0.0

    @pl.when(j == 0)
    def _bce():
        x = dl_ref[...]
        y = lb_ref[...]
        per_elem = (jnp.maximum(x, 0.0) - x * y
                    + jnp.log1p(jnp.exp(-jnp.abs(x))))
        bce_ref[0, 0] += jnp.sum(per_elem)

    @pl.when(j > 0)
    def _sig():
        sig_ref[0, 0] += jnp.sum(jax.nn.sigmoid(cs_ref[...]))


def _tc_combine_body(tk_ref, sc_sig_ref, bce_ref, sig_ref, out_ref):
    probs = jax.nn.sigmoid(tk_ref[...])
    lane = lax.broadcasted_iota(jnp.int32, probs.shape, 1)
    keep = (lane % _LANES) >= (_LANES - _TOPK)
    tk_sum = jnp.sum(jnp.where(keep, probs, 0.0))
    loss_dx = bce_ref[0, 0] / jnp.float32(_ROWS * 1000)
    loss_sparse = ((sig_ref[0, 0] + jnp.sum(sc_sig_ref[...]))
                   / jnp.float32(_ROWS * _COLS))
    tk_avg = tk_sum / jnp.float32(_ROWS * _TOPK)
    out_ref[0] = (_ALPHA * loss_dx + _BETA * loss_sparse
                  - _GAMMA * tk_avg)
    out_ref[1] = loss_dx
    out_ref[2] = loss_sparse
    out_ref[3] = -tk_avg
    out_ref[4] = tk_avg


def kernel(diagnosis_logits, labels, concept_scores):
    tk_scores, sc_sig = _sc_topk()(concept_scores)

    dl_t = diagnosis_logits.T
    lb_t = labels.T
    n_lbl = diagnosis_logits.shape[1]

    bce_sum, sig_sum = pl.pallas_call(
        _tc_dense_body,
        grid=(_GRID, _CGRID),
        in_specs=[
            pl.BlockSpec((n_lbl, _BLOCK_ROWS), lambda i, j: (0, i)),
            pl.BlockSpec((n_lbl, _BLOCK_ROWS), lambda i, j: (0, i)),
            pl.BlockSpec((_BLOCK_ROWS, _CBLK),
                         lambda i, j: (i, jnp.maximum(j, 1))),
        ],
        out_specs=[
            pl.BlockSpec(memory_space=pltpu.SMEM),
            pl.BlockSpec(memory_space=pltpu.SMEM),
        ],
        out_shape=[
            jax.ShapeDtypeStruct((1, 1), jnp.float32),
            jax.ShapeDtypeStruct((1, 1), jnp.float32),
        ],
    )(dl_t, lb_t, concept_scores)

    out = pl.pallas_call(
        _tc_combine_body,
        in_specs=[
            pl.BlockSpec(memory_space=pltpu.VMEM),
            pl.BlockSpec(memory_space=pltpu.VMEM),
            pl.BlockSpec(memory_space=pltpu.SMEM),
            pl.BlockSpec(memory_space=pltpu.SMEM),
        ],
        out_specs=pl.BlockSpec(memory_space=pltpu.SMEM),
        out_shape=jax.ShapeDtypeStruct((5,), jnp.float32),
    )(tk_scores, sc_sig, bce_sum, sig_sum)

    return (out[0], out[1], out[2], out[3], out[4])


# speedup vs baseline: 1.5116x; 1.5116x over previous
import functools

import jax
import jax.numpy as jnp
from jax import lax
from jax.experimental import pallas as pl
from jax.experimental.pallas import tpu as pltpu
from jax.experimental.pallas import tpu_sc as plsc

_ALPHA = 0.6
_BETA = 0.25
_GAMMA = 0.15
_TOPK = 10

_ROWS = 4096
_COLS = 2048
_LANES = 16
_NC = 2
_NS = 16
_NW = _NC * _NS
_ROWS_PER_W = _ROWS // _NW
_ROWS_PER_BLK = 16
_NBLK = _ROWS_PER_W // _ROWS_PER_BLK
_CHUNKS = _COLS // _LANES
_RI = 8
_OCOLS = _ROWS_PER_BLK * _LANES

_BLOCK_ROWS = 256
_GRID = _ROWS // _BLOCK_ROWS
_SIG_CHUNKS = 32
_CBLK = 512
_CGRID = _COLS // _CBLK


def _sigmoid16(x):
    return 1.0 / (1.0 + jnp.exp(-x))


def _sc_body(cs_hbm, tk_out, sig_out, buf, obuf, sem0, sem1):
    wid = lax.axis_index("s") * _NC + lax.axis_index("c")
    row0 = wid * _ROWS_PER_W

    sems = (sem0, sem1)
    neg_inf = jnp.full((_LANES,), -jnp.inf, dtype=jnp.float32)
    sig_acc = jnp.zeros((_LANES,), dtype=jnp.float32)

    copies = [None, None]
    copies[0] = pltpu.async_copy(
        cs_hbm.at[pl.ds(row0, _ROWS_PER_BLK)], buf.at[0], sems[0])
    for blk in range(_NBLK):
        cur = blk % 2
        copies[cur].wait()
        if blk + 1 < _NBLK:
            copies[1 - cur] = pltpu.async_copy(
                cs_hbm.at[pl.ds(row0 + (blk + 1) * _ROWS_PER_BLK,
                                _ROWS_PER_BLK)],
                buf.at[1 - cur], sems[1 - cur])
        for rg in range(_ROWS_PER_BLK // _RI):
            def chunk_step_sig(c, carry):
                ts = carry[:_RI]
                sacc = carry[_RI]
                new_ts = []
                for r in range(_RI):
                    v = buf[cur, rg * _RI + r, pl.ds(c * _LANES, _LANES)]
                    sacc = sacc + _sigmoid16(v)
                    vd, _ = plsc.sort_key_val(v, v, descending=True)
                    tb = jnp.maximum(ts[r], vd)
                    ta, _ = plsc.sort_key_val(tb, tb, descending=False)
                    new_ts.append(ta)
                return tuple(new_ts) + (sacc,)

            def chunk_step(c, ts):
                new_ts = []
                for r in range(_RI):
                    v = buf[cur, rg * _RI + r, pl.ds(c * _LANES, _LANES)]
                    vd, _ = plsc.sort_key_val(v, v, descending=True)
                    tb = jnp.maximum(ts[r], vd)
                    ta, _ = plsc.sort_key_val(tb, tb, descending=False)
                    new_ts.append(ta)
                return tuple(new_ts)

            mid = lax.fori_loop(
                0, _SIG_CHUNKS, chunk_step_sig,
                tuple(neg_inf for _ in range(_RI)) + (sig_acc,))
            sig_acc = mid[_RI]
            out = lax.fori_loop(_SIG_CHUNKS, _CHUNKS, chunk_step, mid[:_RI])
            for r in range(_RI):
                row_local = blk * _ROWS_PER_BLK + rg * _RI + r
                obuf[row_local // _ROWS_PER_BLK,
                     pl.ds((row_local % _ROWS_PER_BLK) * _LANES,
                           _LANES)] = out[r]

    pltpu.sync_copy(obuf, tk_out.at[pl.ds(wid * _NBLK, _NBLK)])
    buf[0, 0, pl.ds(0, _LANES)] = sig_acc
    pltpu.sync_copy(buf.at[0, 0, pl.ds(0, _LANES)], sig_out.at[wid])


@functools.cache
def _sc_topk():
    return pl.kernel(
        _sc_body,
        out_type=[
            jax.ShapeDtypeStruct((_ROWS // _ROWS_PER_BLK, _OCOLS),
                                 jnp.float32),
            jax.ShapeDtypeStruct((_NW, _LANES), jnp.float32),
        ],
        mesh=plsc.VectorSubcoreMesh(
            core_axis_name="c", subcore_axis_name="s",
            num_cores=_NC, num_subcores=_NS),
        scratch_types=[
            pltpu.VMEM((2, _ROWS_PER_BLK, _COLS), jnp.float32),
            pltpu.VMEM((_NBLK, _OCOLS), jnp.float32),
            pltpu.SemaphoreType.DMA,
            pltpu.SemaphoreType.DMA,
        ],
        compiler_params=pltpu.CompilerParams(
            needs_layout_passes=False, use_tc_tiling_on_sc=True),
    )


def _tc_dense_body(dl_ref, lb_ref, cs_a_ref, cs_b_ref, bce_ref, sig_ref):
    @pl.when(pl.program_id(0) == 0)
    def _init():
        bce_ref[0, 0] = 0.0
        sig_ref[0, 0] = 0.0

    x = dl_ref[...]
    y = lb_ref[...]
    per_elem = (jnp.maximum(x, 0.0) - x * y
                + jnp.log1p(jnp.exp(-jnp.abs(x))))
    bce_ref[0, 0] += jnp.sum(per_elem)
    sig_ref[0, 0] += (jnp.sum(jax.nn.sigmoid(cs_a_ref[...]))
                      + jnp.sum(jax.nn.sigmoid(cs_b_ref[...])))


def _tc_combine_body(tk_ref, sc_sig_ref, bce_ref, sig_ref, out_ref):
    probs = jax.nn.sigmoid(tk_ref[...])
    lane = lax.broadcasted_iota(jnp.int32, probs.shape, 1)
    keep = (lane % _LANES) >= (_LANES - _TOPK)
    tk_sum = jnp.sum(jnp.where(keep, probs, 0.0))
    loss_dx = bce_ref[0, 0] / jnp.float32(_ROWS * 1000)
    loss_sparse = ((sig_ref[0, 0] + jnp.sum(sc_sig_ref[...]))
                   / jnp.float32(_ROWS * _COLS))
    tk_avg = tk_sum / jnp.float32(_ROWS * _TOPK)
    out_ref[0] = (_ALPHA * loss_dx + _BETA * loss_sparse
                  - _GAMMA * tk_avg)
    out_ref[1] = loss_dx
    out_ref[2] = loss_sparse
    out_ref[3] = -tk_avg
    out_ref[4] = tk_avg


def kernel(diagnosis_logits, labels, concept_scores):
    tk_scores, sc_sig = _sc_topk()(concept_scores)

    dl_t = diagnosis_logits.T
    lb_t = labels.T
    n_lbl = diagnosis_logits.shape[1]

    bce_sum, sig_sum = pl.pallas_call(
        _tc_dense_body,
        grid=(_GRID,),
        in_specs=[
            pl.BlockSpec((n_lbl, _BLOCK_ROWS), lambda i: (0, i)),
            pl.BlockSpec((n_lbl, _BLOCK_ROWS), lambda i: (0, i)),
            pl.BlockSpec((_BLOCK_ROWS, _CBLK), lambda i: (i, 1)),
            pl.BlockSpec((_BLOCK_ROWS, 2 * _CBLK), lambda i: (i, 1)),
        ],
        out_specs=[
            pl.BlockSpec(memory_space=pltpu.SMEM),
            pl.BlockSpec(memory_space=pltpu.SMEM),
        ],
        out_shape=[
            jax.ShapeDtypeStruct((1, 1), jnp.float32),
            jax.ShapeDtypeStruct((1, 1), jnp.float32),
        ],
    )(dl_t, lb_t, concept_scores, concept_scores)

    out = pl.pallas_call(
        _tc_combine_body,
        in_specs=[
            pl.BlockSpec(memory_space=pltpu.VMEM),
            pl.BlockSpec(memory_space=pltpu.VMEM),
            pl.BlockSpec(memory_space=pltpu.SMEM),
            pl.BlockSpec(memory_space=pltpu.SMEM),
        ],
        out_specs=pl.BlockSpec(memory_space=pltpu.SMEM),
        out_shape=jax.ShapeDtypeStruct((5,), jnp.float32),
    )(tk_scores, sc_sig, bce_sum, sig_sum)

    return (out[0], out[1], out[2], out[3], out[4])


# speedup vs baseline: 1.6167x; 1.0695x over previous
import functools

import jax
import jax.numpy as jnp
from jax import lax
from jax.experimental import pallas as pl
from jax.experimental.pallas import tpu as pltpu
from jax.experimental.pallas import tpu_sc as plsc

_ALPHA = 0.6
_BETA = 0.25
_GAMMA = 0.15
_TOPK = 10

_ROWS = 4096
_COLS = 2048
_LANES = 16
_NC = 2
_NS = 16
_NW = _NC * _NS
_ROWS_PER_W = _ROWS // _NW
_ROWS_PER_BLK = 16
_NBLK = _ROWS_PER_W // _ROWS_PER_BLK
_CHUNKS = _COLS // _LANES
_RI = 16
_OCOLS = _ROWS_PER_BLK * _LANES

_BLOCK_ROWS = 256
_GRID = _ROWS // _BLOCK_ROWS
_SIG_CHUNKS = 32
_CBLK = 512
_CGRID = _COLS // _CBLK


def _sigmoid16(x):
    return 1.0 / (1.0 + jnp.exp(-x))


def _sc_body(cs_hbm, tk_out, sig_out, buf, obuf, sem0, sem1):
    wid = lax.axis_index("s") * _NC + lax.axis_index("c")
    row0 = wid * _ROWS_PER_W

    sems = (sem0, sem1)
    neg_inf = jnp.full((_LANES,), -jnp.inf, dtype=jnp.float32)
    sig_acc = jnp.zeros((_LANES,), dtype=jnp.float32)

    copies = [None, None]
    copies[0] = pltpu.async_copy(
        cs_hbm.at[pl.ds(row0, _ROWS_PER_BLK)], buf.at[0], sems[0])
    for blk in range(_NBLK):
        cur = blk % 2
        copies[cur].wait()
        if blk + 1 < _NBLK:
            copies[1 - cur] = pltpu.async_copy(
                cs_hbm.at[pl.ds(row0 + (blk + 1) * _ROWS_PER_BLK,
                                _ROWS_PER_BLK)],
                buf.at[1 - cur], sems[1 - cur])
        for rg in range(_ROWS_PER_BLK // _RI):
            def chunk_step_sig(c, carry):
                ts = carry[:_RI]
                sacc = carry[_RI]
                new_ts = []
                for r in range(_RI):
                    v = buf[cur, rg * _RI + r, pl.ds(c * _LANES, _LANES)]
                    sacc = sacc + _sigmoid16(v)
                    vd, _ = plsc.sort_key_val(v, v, descending=True)
                    tb = jnp.maximum(ts[r], vd)
                    ta, _ = plsc.sort_key_val(tb, tb, descending=False)
                    new_ts.append(ta)
                return tuple(new_ts) + (sacc,)

            def chunk_step(c, ts):
                new_ts = []
                for r in range(_RI):
                    v = buf[cur, rg * _RI + r, pl.ds(c * _LANES, _LANES)]
                    vd, _ = plsc.sort_key_val(v, v, descending=True)
                    tb = jnp.maximum(ts[r], vd)
                    ta, _ = plsc.sort_key_val(tb, tb, descending=False)
                    new_ts.append(ta)
                return tuple(new_ts)

            mid = lax.fori_loop(
                0, _SIG_CHUNKS, chunk_step_sig,
                tuple(neg_inf for _ in range(_RI)) + (sig_acc,))
            sig_acc = mid[_RI]
            out = lax.fori_loop(_SIG_CHUNKS, _CHUNKS, chunk_step, mid[:_RI])
            for r in range(_RI):
                row_local = blk * _ROWS_PER_BLK + rg * _RI + r
                obuf[row_local // _ROWS_PER_BLK,
                     pl.ds((row_local % _ROWS_PER_BLK) * _LANES,
                           _LANES)] = out[r]

    pltpu.sync_copy(obuf, tk_out.at[pl.ds(wid * _NBLK, _NBLK)])
    buf[0, 0, pl.ds(0, _LANES)] = sig_acc
    pltpu.sync_copy(buf.at[0, 0, pl.ds(0, _LANES)], sig_out.at[wid])


@functools.cache
def _sc_topk():
    return pl.kernel(
        _sc_body,
        out_type=[
            jax.ShapeDtypeStruct((_ROWS // _ROWS_PER_BLK, _OCOLS),
                                 jnp.float32),
            jax.ShapeDtypeStruct((_NW, _LANES), jnp.float32),
        ],
        mesh=plsc.VectorSubcoreMesh(
            core_axis_name="c", subcore_axis_name="s",
            num_cores=_NC, num_subcores=_NS),
        scratch_types=[
            pltpu.VMEM((2, _ROWS_PER_BLK, _COLS), jnp.float32),
            pltpu.VMEM((_NBLK, _OCOLS), jnp.float32),
            pltpu.SemaphoreType.DMA,
            pltpu.SemaphoreType.DMA,
        ],
        compiler_params=pltpu.CompilerParams(
            needs_layout_passes=False, use_tc_tiling_on_sc=True),
    )


def _tc_dense_body(dl_ref, lb_ref, cs_a_ref, cs_b_ref, bce_ref, sig_ref):
    @pl.when(pl.program_id(0) == 0)
    def _init():
        bce_ref[0, 0] = 0.0
        sig_ref[0, 0] = 0.0

    x = dl_ref[...]
    y = lb_ref[...]
    per_elem = (jnp.maximum(x, 0.0) - x * y
                + jnp.log1p(jnp.exp(-jnp.abs(x))))
    bce_ref[0, 0] += jnp.sum(per_elem)
    sig_ref[0, 0] += (jnp.sum(jax.nn.sigmoid(cs_a_ref[...]))
                      + jnp.sum(jax.nn.sigmoid(cs_b_ref[...])))


def _tc_combine_body(tk_ref, sc_sig_ref, bce_ref, sig_ref, out_ref):
    probs = jax.nn.sigmoid(tk_ref[...])
    lane = lax.broadcasted_iota(jnp.int32, probs.shape, 1)
    keep = (lane % _LANES) >= (_LANES - _TOPK)
    tk_sum = jnp.sum(jnp.where(keep, probs, 0.0))
    loss_dx = bce_ref[0, 0] / jnp.float32(_ROWS * 1000)
    loss_sparse = ((sig_ref[0, 0] + jnp.sum(sc_sig_ref[...]))
                   / jnp.float32(_ROWS * _COLS))
    tk_avg = tk_sum / jnp.float32(_ROWS * _TOPK)
    out_ref[0] = (_ALPHA * loss_dx + _BETA * loss_sparse
                  - _GAMMA * tk_avg)
    out_ref[1] = loss_dx
    out_ref[2] = loss_sparse
    out_ref[3] = -tk_avg
    out_ref[4] = tk_avg


def kernel(diagnosis_logits, labels, concept_scores):
    tk_scores, sc_sig = _sc_topk()(concept_scores)

    dl_t = diagnosis_logits.T
    lb_t = labels.T
    n_lbl = diagnosis_logits.shape[1]

    bce_sum, sig_sum = pl.pallas_call(
        _tc_dense_body,
        grid=(_GRID,),
        in_specs=[
            pl.BlockSpec((n_lbl, _BLOCK_ROWS), lambda i: (0, i)),
            pl.BlockSpec((n_lbl, _BLOCK_ROWS), lambda i: (0, i)),
            pl.BlockSpec((_BLOCK_ROWS, _CBLK), lambda i: (i, 1)),
            pl.BlockSpec((_BLOCK_ROWS, 2 * _CBLK), lambda i: (i, 1)),
        ],
        out_specs=[
            pl.BlockSpec(memory_space=pltpu.SMEM),
            pl.BlockSpec(memory_space=pltpu.SMEM),
        ],
        out_shape=[
            jax.ShapeDtypeStruct((1, 1), jnp.float32),
            jax.ShapeDtypeStruct((1, 1), jnp.float32),
        ],
    )(dl_t, lb_t, concept_scores, concept_scores)

    out = pl.pallas_call(
        _tc_combine_body,
        in_specs=[
            pl.BlockSpec(memory_space=pltpu.VMEM),
            pl.BlockSpec(memory_space=pltpu.VMEM),
            pl.BlockSpec(memory_space=pltpu.SMEM),
            pl.BlockSpec(memory_space=pltpu.SMEM),
        ],
        out_specs=pl.BlockSpec(memory_space=pltpu.SMEM),
        out_shape=jax.ShapeDtypeStruct((5,), jnp.float32),
    )(tk_scores, sc_sig, bce_sum, sig_sum)

    return (out[0], out[1], out[2], out[3], out[4])


# speedup vs baseline: 1.6745x; 1.0357x over previous
import functools

import jax
import jax.numpy as jnp
from jax import lax
from jax.experimental import pallas as pl
from jax.experimental.pallas import tpu as pltpu
from jax.experimental.pallas import tpu_sc as plsc

_ALPHA = 0.6
_BETA = 0.25
_GAMMA = 0.15
_TOPK = 10

_ROWS = 4096
_COLS = 2048
_LANES = 16
_NC = 2
_NS = 16
_NW = _NC * _NS
_ROWS_PER_W = _ROWS // _NW
_ROWS_PER_BLK = 16
_NBLK = _ROWS_PER_W // _ROWS_PER_BLK
_CHUNKS = _COLS // _LANES
_RI = 16
_OCOLS = _ROWS_PER_BLK * _LANES

_BLOCK_ROWS = 256
_GRID = _ROWS // _BLOCK_ROWS
_SIG_CHUNKS = 32
_CBLK = 512
_CGRID = _COLS // _CBLK


def _sigmoid16(x):
    return 1.0 / (1.0 + jnp.exp(-x))


def _sc_body(cs_hbm, tk_out, sig_out, buf, obuf, sem0, sem1):
    wid = lax.axis_index("s") * _NC + lax.axis_index("c")
    row0 = wid * _ROWS_PER_W

    sems = (sem0, sem1)
    neg_inf = jnp.full((_LANES,), -jnp.inf, dtype=jnp.float32)
    sig_acc = jnp.zeros((_LANES,), dtype=jnp.float32)

    pltpu.async_copy(
        cs_hbm.at[pl.ds(row0, _ROWS_PER_BLK)], buf.at[0], sems[0])

    def blk_pair(g, sig_acc_c):
        for b in range(2):
            blk = g * 2 + b
            pltpu.make_async_copy(
                cs_hbm.at[pl.ds(row0, _ROWS_PER_BLK)], buf.at[b],
                sems[b]).wait()

            @pl.when(blk + 1 < _NBLK)
            def _next():
                pltpu.async_copy(
                    cs_hbm.at[pl.ds(row0 + (blk + 1) * _ROWS_PER_BLK,
                                    _ROWS_PER_BLK)],
                    buf.at[1 - b], sems[1 - b])

            def chunk_step_sig(c, carry):
                ts = carry[:_RI]
                sacc = carry[_RI]
                new_ts = []
                for r in range(_RI):
                    v = buf[b, r, pl.ds(c * _LANES, _LANES)]
                    sacc = sacc + _sigmoid16(v)
                    vd, _ = plsc.sort_key_val(v, v, descending=True)
                    tb = jnp.maximum(ts[r], vd)
                    ta, _ = plsc.sort_key_val(tb, tb, descending=False)
                    new_ts.append(ta)
                return tuple(new_ts) + (sacc,)

            def chunk_step(c, ts):
                new_ts = []
                for r in range(_RI):
                    v = buf[b, r, pl.ds(c * _LANES, _LANES)]
                    vd, _ = plsc.sort_key_val(v, v, descending=True)
                    tb = jnp.maximum(ts[r], vd)
                    ta, _ = plsc.sort_key_val(tb, tb, descending=False)
                    new_ts.append(ta)
                return tuple(new_ts)

            mid = lax.fori_loop(
                0, _SIG_CHUNKS, chunk_step_sig,
                tuple(neg_inf for _ in range(_RI)) + (sig_acc_c,))
            sig_acc_c = mid[_RI]
            out = lax.fori_loop(_SIG_CHUNKS, _CHUNKS, chunk_step, mid[:_RI])
            for r in range(_RI):
                obuf[blk, pl.ds(r * _LANES, _LANES)] = out[r]
        return sig_acc_c

    sig_acc = lax.fori_loop(0, _NBLK // 2, blk_pair, sig_acc)

    pltpu.sync_copy(obuf, tk_out.at[pl.ds(wid * _NBLK, _NBLK)])
    buf[0, 0, pl.ds(0, _LANES)] = sig_acc
    pltpu.sync_copy(buf.at[0, 0, pl.ds(0, _LANES)], sig_out.at[wid])


@functools.cache
def _sc_topk():
    return pl.kernel(
        _sc_body,
        out_type=[
            jax.ShapeDtypeStruct((_ROWS // _ROWS_PER_BLK, _OCOLS),
                                 jnp.float32),
            jax.ShapeDtypeStruct((_NW, _LANES), jnp.float32),
        ],
        mesh=plsc.VectorSubcoreMesh(
            core_axis_name="c", subcore_axis_name="s",
            num_cores=_NC, num_subcores=_NS),
        scratch_types=[
            pltpu.VMEM((2, _ROWS_PER_BLK, _COLS), jnp.float32),
            pltpu.VMEM((_NBLK, _OCOLS), jnp.float32),
            pltpu.SemaphoreType.DMA,
            pltpu.SemaphoreType.DMA,
        ],
        compiler_params=pltpu.CompilerParams(
            needs_layout_passes=False, use_tc_tiling_on_sc=True),
    )


def _tc_dense_body(dl_ref, lb_ref, cs_a_ref, cs_b_ref, bce_ref, sig_ref):
    @pl.when(pl.program_id(0) == 0)
    def _init():
        bce_ref[0, 0] = 0.0
        sig_ref[0, 0] = 0.0

    x = dl_ref[...]
    y = lb_ref[...]
    per_elem = (jnp.maximum(x, 0.0) - x * y
                + jnp.log1p(jnp.exp(-jnp.abs(x))))
    bce_ref[0, 0] += jnp.sum(per_elem)
    sig_ref[0, 0] += (jnp.sum(jax.nn.sigmoid(cs_a_ref[...]))
                      + jnp.sum(jax.nn.sigmoid(cs_b_ref[...])))


def _tc_combine_body(tk_ref, sc_sig_ref, bce_ref, sig_ref, out_ref):
    probs = jax.nn.sigmoid(tk_ref[...])
    lane = lax.broadcasted_iota(jnp.int32, probs.shape, 1)
    keep = (lane % _LANES) >= (_LANES - _TOPK)
    tk_sum = jnp.sum(jnp.where(keep, probs, 0.0))
    loss_dx = bce_ref[0, 0] / jnp.float32(_ROWS * 1000)
    loss_sparse = ((sig_ref[0, 0] + jnp.sum(sc_sig_ref[...]))
                   / jnp.float32(_ROWS * _COLS))
    tk_avg = tk_sum / jnp.float32(_ROWS * _TOPK)
    out_ref[0] = (_ALPHA * loss_dx + _BETA * loss_sparse
                  - _GAMMA * tk_avg)
    out_ref[1] = loss_dx
    out_ref[2] = loss_sparse
    out_ref[3] = -tk_avg
    out_ref[4] = tk_avg


def kernel(diagnosis_logits, labels, concept_scores):
    tk_scores, sc_sig = _sc_topk()(concept_scores)

    dl_t = diagnosis_logits.T
    lb_t = labels.T
    n_lbl = diagnosis_logits.shape[1]

    bce_sum, sig_sum = pl.pallas_call(
        _tc_dense_body,
        grid=(_GRID,),
        in_specs=[
            pl.BlockSpec((n_lbl, _BLOCK_ROWS), lambda i: (0, i)),
            pl.BlockSpec((n_lbl, _BLOCK_ROWS), lambda i: (0, i)),
            pl.BlockSpec((_BLOCK_ROWS, _CBLK), lambda i: (i, 1)),
            pl.BlockSpec((_BLOCK_ROWS, 2 * _CBLK), lambda i: (i, 1)),
        ],
        out_specs=[
            pl.BlockSpec(memory_space=pltpu.SMEM),
            pl.BlockSpec(memory_space=pltpu.SMEM),
        ],
        out_shape=[
            jax.ShapeDtypeStruct((1, 1), jnp.float32),
            jax.ShapeDtypeStruct((1, 1), jnp.float32),
        ],
    )(dl_t, lb_t, concept_scores, concept_scores)

    out = pl.pallas_call(
        _tc_combine_body,
        in_specs=[
            pl.BlockSpec(memory_space=pltpu.VMEM),
            pl.BlockSpec(memory_space=pltpu.VMEM),
            pl.BlockSpec(memory_space=pltpu.SMEM),
            pl.BlockSpec(memory_space=pltpu.SMEM),
        ],
        out_specs=pl.BlockSpec(memory_space=pltpu.SMEM),
        out_shape=jax.ShapeDtypeStruct((5,), jnp.float32),
    )(tk_scores, sc_sig, bce_sum, sig_sum)

    return (out[0], out[1], out[2], out[3], out[4])


# speedup vs baseline: 1.6756x; 1.0007x over previous
import functools

import jax
import jax.numpy as jnp
from jax import lax
from jax.experimental import pallas as pl
from jax.experimental.pallas import tpu as pltpu
from jax.experimental.pallas import tpu_sc as plsc

_ALPHA = 0.6
_BETA = 0.25
_GAMMA = 0.15
_TOPK = 10

_ROWS = 4096
_COLS = 2048
_LANES = 16
_NC = 2
_NS = 16
_NW = _NC * _NS
_ROWS_PER_W = _ROWS // _NW
_ROWS_PER_BLK = 16
_NBLK = _ROWS_PER_W // _ROWS_PER_BLK
_CHUNKS = _COLS // _LANES
_RI = 16
_OCOLS = _ROWS_PER_BLK * _LANES

_BLOCK_ROWS = 256
_GRID = _ROWS // _BLOCK_ROWS
_SIG_CHUNKS = 32
_CBLK = 512
_CGRID = _COLS // _CBLK


def _sigmoid16(x):
    return 1.0 / (1.0 + jnp.exp(-x))


def _sc_body(cs_hbm, tk_out, sig_out, buf, obuf, sem0, sem1):
    wid = lax.axis_index("s") * _NC + lax.axis_index("c")
    row0 = wid * _ROWS_PER_W

    sems = (sem0, sem1)
    neg_inf = jnp.full((_LANES,), -jnp.inf, dtype=jnp.float32)
    sig_acc = jnp.zeros((_LANES,), dtype=jnp.float32)

    pltpu.async_copy(
        cs_hbm.at[pl.ds(row0, _ROWS_PER_BLK)], buf.at[0], sems[0])

    def blk_pair(g, sig_acc_c):
        for b in range(2):
            blk = g * 2 + b
            pltpu.make_async_copy(
                cs_hbm.at[pl.ds(row0, _ROWS_PER_BLK)], buf.at[b],
                sems[b]).wait()

            @pl.when(blk + 1 < _NBLK)
            def _next():
                pltpu.async_copy(
                    cs_hbm.at[pl.ds(row0 + (blk + 1) * _ROWS_PER_BLK,
                                    _ROWS_PER_BLK)],
                    buf.at[1 - b], sems[1 - b])

            def chunk_step_sig(c, carry):
                ts = carry[:_RI]
                sacc = carry[_RI]
                new_ts = []
                for r in range(_RI):
                    v = buf[b, r, pl.ds(c * _LANES, _LANES)]
                    sacc = sacc + _sigmoid16(v)
                    vd, _ = plsc.sort_key_val(v, v, descending=True)
                    tb = jnp.maximum(ts[r], vd)
                    ta, _ = plsc.sort_key_val(tb, tb, descending=False)
                    new_ts.append(ta)
                return tuple(new_ts) + (sacc,)

            def chunk_step(c, ts):
                new_ts = []
                for r in range(_RI):
                    v = buf[b, r, pl.ds(c * _LANES, _LANES)]
                    vd, _ = plsc.sort_key_val(v, v, descending=True)
                    tb = jnp.maximum(ts[r], vd)
                    ta, _ = plsc.sort_key_val(tb, tb, descending=False)
                    new_ts.append(ta)
                return tuple(new_ts)

            mid = lax.fori_loop(
                0, _SIG_CHUNKS, chunk_step_sig,
                tuple(neg_inf for _ in range(_RI)) + (sig_acc_c,))
            sig_acc_c = mid[_RI]
            out = lax.fori_loop(_SIG_CHUNKS, _CHUNKS, chunk_step, mid[:_RI])
            for r in range(_RI):
                obuf[blk, pl.ds(r * _LANES, _LANES)] = out[r]
        return sig_acc_c

    sig_acc = lax.fori_loop(0, _NBLK // 2, blk_pair, sig_acc)

    pltpu.sync_copy(obuf, tk_out.at[pl.ds(wid * _NBLK, _NBLK)])
    buf[0, 0, pl.ds(0, _LANES)] = sig_acc
    pltpu.sync_copy(buf.at[0, 0, pl.ds(0, _LANES)], sig_out.at[wid])


@functools.cache
def _sc_topk():
    return pl.kernel(
        _sc_body,
        out_type=[
            jax.ShapeDtypeStruct((_ROWS // _ROWS_PER_BLK, _OCOLS),
                                 jnp.float32),
            jax.ShapeDtypeStruct((_NW, _LANES), jnp.float32),
        ],
        mesh=plsc.VectorSubcoreMesh(
            core_axis_name="c", subcore_axis_name="s",
            num_cores=_NC, num_subcores=_NS),
        scratch_types=[
            pltpu.VMEM((2, _ROWS_PER_BLK, _COLS), jnp.float32),
            pltpu.VMEM((_NBLK, _OCOLS), jnp.float32),
            pltpu.SemaphoreType.DMA,
            pltpu.SemaphoreType.DMA,
        ],
        compiler_params=pltpu.CompilerParams(
            needs_layout_passes=False, use_tc_tiling_on_sc=True,
            skip_device_barrier=True),
    )


def _tc_dense_body(dl_ref, lb_ref, cs_a_ref, cs_b_ref, bce_ref, sig_ref):
    @pl.when(pl.program_id(0) == 0)
    def _init():
        bce_ref[0, 0] = 0.0
        sig_ref[0, 0] = 0.0

    x = dl_ref[...]
    y = lb_ref[...]
    per_elem = (jnp.maximum(x, 0.0) - x * y
                + jnp.log1p(jnp.exp(-jnp.abs(x))))
    bce_ref[0, 0] += jnp.sum(per_elem)
    sig_ref[0, 0] += (jnp.sum(jax.nn.sigmoid(cs_a_ref[...]))
                      + jnp.sum(jax.nn.sigmoid(cs_b_ref[...])))


def _tc_combine_body(tk_ref, sc_sig_ref, bce_ref, sig_ref, out_ref):
    probs = jax.nn.sigmoid(tk_ref[...])
    lane = lax.broadcasted_iota(jnp.int32, probs.shape, 1)
    keep = (lane % _LANES) >= (_LANES - _TOPK)
    tk_sum = jnp.sum(jnp.where(keep, probs, 0.0))
    loss_dx = bce_ref[0, 0] / jnp.float32(_ROWS * 1000)
    loss_sparse = ((sig_ref[0, 0] + jnp.sum(sc_sig_ref[...]))
                   / jnp.float32(_ROWS * _COLS))
    tk_avg = tk_sum / jnp.float32(_ROWS * _TOPK)
    out_ref[0] = (_ALPHA * loss_dx + _BETA * loss_sparse
                  - _GAMMA * tk_avg)
    out_ref[1] = loss_dx
    out_ref[2] = loss_sparse
    out_ref[3] = -tk_avg
    out_ref[4] = tk_avg


def kernel(diagnosis_logits, labels, concept_scores):
    tk_scores, sc_sig = _sc_topk()(concept_scores)

    dl_t = diagnosis_logits.T
    lb_t = labels.T
    n_lbl = diagnosis_logits.shape[1]

    bce_sum, sig_sum = pl.pallas_call(
        _tc_dense_body,
        grid=(_GRID,),
        in_specs=[
            pl.BlockSpec((n_lbl, _BLOCK_ROWS), lambda i: (0, i)),
            pl.BlockSpec((n_lbl, _BLOCK_ROWS), lambda i: (0, i)),
            pl.BlockSpec((_BLOCK_ROWS, _CBLK), lambda i: (i, 1)),
            pl.BlockSpec((_BLOCK_ROWS, 2 * _CBLK), lambda i: (i, 1)),
        ],
        out_specs=[
            pl.BlockSpec(memory_space=pltpu.SMEM),
            pl.BlockSpec(memory_space=pltpu.SMEM),
        ],
        out_shape=[
            jax.ShapeDtypeStruct((1, 1), jnp.float32),
            jax.ShapeDtypeStruct((1, 1), jnp.float32),
        ],
    )(dl_t, lb_t, concept_scores, concept_scores)

    out = pl.pallas_call(
        _tc_combine_body,
        in_specs=[
            pl.BlockSpec(memory_space=pltpu.VMEM),
            pl.BlockSpec(memory_space=pltpu.VMEM),
            pl.BlockSpec(memory_space=pltpu.SMEM),
            pl.BlockSpec(memory_space=pltpu.SMEM),
        ],
        out_specs=pl.BlockSpec(memory_space=pltpu.SMEM),
        out_shape=jax.ShapeDtypeStruct((5,), jnp.float32),
    )(tk_scores, sc_sig, bce_sum, sig_sum)

    return (out[0], out[1], out[2], out[3], out[4])


# speedup vs baseline: 1.7250x; 1.0294x over previous
import functools

import jax
import jax.numpy as jnp
from jax import lax
from jax.experimental import pallas as pl
from jax.experimental.pallas import tpu as pltpu
from jax.experimental.pallas import tpu_sc as plsc

_ALPHA = 0.6
_BETA = 0.25
_GAMMA = 0.15
_TOPK = 10

_ROWS = 4096
_COLS = 2048
_LANES = 16
_NC = 2
_NS = 16
_NW = _NC * _NS
_ROWS_PER_W = _ROWS // _NW
_ROWS_PER_BLK = 16
_NBLK = _ROWS_PER_W // _ROWS_PER_BLK
_CHUNKS = _COLS // _LANES
_RI = 16
_OCOLS = _ROWS_PER_BLK * _LANES

_BLOCK_ROWS = 512
_GRID = _ROWS // _BLOCK_ROWS
_SIG_CHUNKS = 32
_CBLK = 512
_CGRID = _COLS // _CBLK


def _sigmoid16(x):
    return 1.0 / (1.0 + jnp.exp(-x))


def _sc_body(cs_hbm, tk_out, sig_out, buf, obuf, sem0, sem1):
    wid = lax.axis_index("s") * _NC + lax.axis_index("c")
    row0 = wid * _ROWS_PER_W

    sems = (sem0, sem1)
    neg_inf = jnp.full((_LANES,), -jnp.inf, dtype=jnp.float32)
    sig_acc = jnp.zeros((_LANES,), dtype=jnp.float32)

    pltpu.async_copy(
        cs_hbm.at[pl.ds(row0, _ROWS_PER_BLK)], buf.at[0], sems[0])

    def blk_pair(g, sig_acc_c):
        for b in range(2):
            blk = g * 2 + b
            pltpu.make_async_copy(
                cs_hbm.at[pl.ds(row0, _ROWS_PER_BLK)], buf.at[b],
                sems[b]).wait()

            @pl.when(blk + 1 < _NBLK)
            def _next():
                pltpu.async_copy(
                    cs_hbm.at[pl.ds(row0 + (blk + 1) * _ROWS_PER_BLK,
                                    _ROWS_PER_BLK)],
                    buf.at[1 - b], sems[1 - b])

            def chunk_step_sig(c, carry):
                ts = carry[:_RI]
                sacc = carry[_RI]
                new_ts = []
                for r in range(_RI):
                    v = buf[b, r, pl.ds(c * _LANES, _LANES)]
                    sacc = sacc + _sigmoid16(v)
                    vd, _ = plsc.sort_key_val(v, v, descending=True)
                    tb = jnp.maximum(ts[r], vd)
                    ta, _ = plsc.sort_key_val(tb, tb, descending=False)
                    new_ts.append(ta)
                return tuple(new_ts) + (sacc,)

            def chunk_step(c, ts):
                new_ts = []
                for r in range(_RI):
                    v = buf[b, r, pl.ds(c * _LANES, _LANES)]
                    vd, _ = plsc.sort_key_val(v, v, descending=True)
                    tb = jnp.maximum(ts[r], vd)
                    ta, _ = plsc.sort_key_val(tb, tb, descending=False)
                    new_ts.append(ta)
                return tuple(new_ts)

            mid = lax.fori_loop(
                0, _SIG_CHUNKS, chunk_step_sig,
                tuple(neg_inf for _ in range(_RI)) + (sig_acc_c,))
            sig_acc_c = mid[_RI]
            out = lax.fori_loop(_SIG_CHUNKS, _CHUNKS, chunk_step, mid[:_RI])
            for r in range(_RI):
                obuf[blk, pl.ds(r * _LANES, _LANES)] = out[r]
        return sig_acc_c

    sig_acc = lax.fori_loop(0, _NBLK // 2, blk_pair, sig_acc)

    pltpu.sync_copy(obuf, tk_out.at[pl.ds(wid * _NBLK, _NBLK)])
    buf[0, 0, pl.ds(0, _LANES)] = sig_acc
    pltpu.sync_copy(buf.at[0, 0, pl.ds(0, _LANES)], sig_out.at[wid])


@functools.cache
def _sc_topk():
    return pl.kernel(
        _sc_body,
        out_type=[
            jax.ShapeDtypeStruct((_ROWS // _ROWS_PER_BLK, _OCOLS),
                                 jnp.float32),
            jax.ShapeDtypeStruct((_NW, _LANES), jnp.float32),
        ],
        mesh=plsc.VectorSubcoreMesh(
            core_axis_name="c", subcore_axis_name="s",
            num_cores=_NC, num_subcores=_NS),
        scratch_types=[
            pltpu.VMEM((2, _ROWS_PER_BLK, _COLS), jnp.float32),
            pltpu.VMEM((_NBLK, _OCOLS), jnp.float32),
            pltpu.SemaphoreType.DMA,
            pltpu.SemaphoreType.DMA,
        ],
        compiler_params=pltpu.CompilerParams(
            needs_layout_passes=False, use_tc_tiling_on_sc=True),
    )


def _tc_dense_body(dl_ref, lb_ref, cs_a_ref, cs_b_ref, bce_ref, sig_ref):
    @pl.when(pl.program_id(0) == 0)
    def _init():
        bce_ref[0, 0] = 0.0
        sig_ref[0, 0] = 0.0

    x = dl_ref[...]
    y = lb_ref[...]
    per_elem = (jnp.maximum(x, 0.0) - x * y
                + jnp.log1p(jnp.exp(-jnp.abs(x))))
    bce_ref[0, 0] += jnp.sum(per_elem)
    sig_ref[0, 0] += (jnp.sum(jax.nn.sigmoid(cs_a_ref[...]))
                      + jnp.sum(jax.nn.sigmoid(cs_b_ref[...])))


def _tc_combine_body(tk_ref, sc_sig_ref, bce_ref, sig_ref, out_ref):
    probs = jax.nn.sigmoid(tk_ref[...])
    lane = lax.broadcasted_iota(jnp.int32, probs.shape, 1)
    keep = (lane % _LANES) >= (_LANES - _TOPK)
    tk_sum = jnp.sum(jnp.where(keep, probs, 0.0))
    loss_dx = bce_ref[0, 0] / jnp.float32(_ROWS * 1000)
    loss_sparse = ((sig_ref[0, 0] + jnp.sum(sc_sig_ref[...]))
                   / jnp.float32(_ROWS * _COLS))
    tk_avg = tk_sum / jnp.float32(_ROWS * _TOPK)
    out_ref[0] = (_ALPHA * loss_dx + _BETA * loss_sparse
                  - _GAMMA * tk_avg)
    out_ref[1] = loss_dx
    out_ref[2] = loss_sparse
    out_ref[3] = -tk_avg
    out_ref[4] = tk_avg


def kernel(diagnosis_logits, labels, concept_scores):
    tk_scores, sc_sig = _sc_topk()(concept_scores)

    dl_t = diagnosis_logits.T
    lb_t = labels.T
    n_lbl = diagnosis_logits.shape[1]

    bce_sum, sig_sum = pl.pallas_call(
        _tc_dense_body,
        grid=(_GRID,),
        in_specs=[
            pl.BlockSpec((n_lbl, _BLOCK_ROWS), lambda i: (0, i)),
            pl.BlockSpec((n_lbl, _BLOCK_ROWS), lambda i: (0, i)),
            pl.BlockSpec((_BLOCK_ROWS, _CBLK), lambda i: (i, 1)),
            pl.BlockSpec((_BLOCK_ROWS, 2 * _CBLK), lambda i: (i, 1)),
        ],
        out_specs=[
            pl.BlockSpec(memory_space=pltpu.SMEM),
            pl.BlockSpec(memory_space=pltpu.SMEM),
        ],
        out_shape=[
            jax.ShapeDtypeStruct((1, 1), jnp.float32),
            jax.ShapeDtypeStruct((1, 1), jnp.float32),
        ],
    )(dl_t, lb_t, concept_scores, concept_scores)

    out = pl.pallas_call(
        _tc_combine_body,
        in_specs=[
            pl.BlockSpec(memory_space=pltpu.VMEM),
            pl.BlockSpec(memory_space=pltpu.VMEM),
            pl.BlockSpec(memory_space=pltpu.SMEM),
            pl.BlockSpec(memory_space=pltpu.SMEM),
        ],
        out_specs=pl.BlockSpec(memory_space=pltpu.SMEM),
        out_shape=jax.ShapeDtypeStruct((5,), jnp.float32),
    )(tk_scores, sc_sig, bce_sum, sig_sum)

    return (out[0], out[1], out[2], out[3], out[4])
